# Initial kernel scaffold; baseline (speedup 1.0000x reference)
import jax
import jax.numpy as jnp
from jax.experimental import pallas as pl


def kernel(user, item_i, item_j, user_item_3, item_user_3, user_js, embed_user_weight, embed_item_weight, user_item_matrix, item_user_matrix):
    raise NotImplementedError("write your pallas kernel here")


# speedup vs baseline: 1.1209x; 1.1209x over previous
import functools

import jax
import jax.numpy as jnp
from jax import lax
from jax.experimental import pallas as pl
from jax.experimental.pallas import tpu as pltpu
from jax.experimental.pallas import tpu_sc as plsc


_BM = 512


def _mm_body(a_ref, x_ref, o_ref):
    o_ref[...] = jnp.dot(
        a_ref[...].astype(jnp.bfloat16), x_ref[...],
        preferred_element_type=jnp.float32)


def _mm(a, x_bf16, bm=_BM):
    m, k = a.shape
    n = x_bf16.shape[1]
    return pl.pallas_call(
        _mm_body,
        grid=(m // bm,),
        in_specs=[
            pl.BlockSpec((bm, k), lambda i: (i, 0)),
            pl.BlockSpec((k, n), lambda i: (0, 0)),
        ],
        out_specs=pl.BlockSpec((bm, n), lambda i: (i, 0)),
        out_shape=jax.ShapeDtypeStruct((m, n), jnp.float32),
    )(a, x_bf16)


def _mm_users_body(a_ref, x_ref, ue_ref, g1_ref, g2_ref, js_ref, o_ref):
    g3 = jnp.dot(
        a_ref[...].astype(jnp.bfloat16), x_ref[...],
        preferred_element_type=jnp.float32)
    o_ref[...] = (0.25 * (ue_ref[...] + g1_ref[...] + g2_ref[...])
                  + g3 * js_ref[...])


def _mm_users(a, x_bf16, ue, g1, g2, js, bm=_BM):
    m, k = a.shape
    n = x_bf16.shape[1]
    return pl.pallas_call(
        _mm_users_body,
        grid=(m // bm,),
        in_specs=[
            pl.BlockSpec((bm, k), lambda i: (i, 0)),
            pl.BlockSpec((k, n), lambda i: (0, 0)),
            pl.BlockSpec((bm, n), lambda i: (i, 0)),
            pl.BlockSpec((bm, n), lambda i: (i, 0)),
            pl.BlockSpec((bm, n), lambda i: (i, 0)),
            pl.BlockSpec((bm, 1), lambda i: (i, 0)),
        ],
        out_specs=pl.BlockSpec((bm, n), lambda i: (i, 0)),
        out_shape=jax.ShapeDtypeStruct((m, n), jnp.float32),
    )(a, x_bf16, ue, g1, g2, js)


def _mm_items_body(a_ref, x_ref, ie_ref, g1_ref, g2_ref, o_ref):
    g3 = jnp.dot(
        a_ref[...].astype(jnp.bfloat16), x_ref[...],
        preferred_element_type=jnp.float32)
    o_ref[...] = 0.25 * (ie_ref[...] + g1_ref[...] + g2_ref[...] + g3)


def _mm_items(a, x_bf16, ie, g1, g2, bm=_BM):
    m, k = a.shape
    n = x_bf16.shape[1]
    return pl.pallas_call(
        _mm_items_body,
        grid=(m // bm,),
        in_specs=[
            pl.BlockSpec((bm, k), lambda i: (i, 0)),
            pl.BlockSpec((k, n), lambda i: (0, 0)),
            pl.BlockSpec((bm, n), lambda i: (i, 0)),
            pl.BlockSpec((bm, n), lambda i: (i, 0)),
            pl.BlockSpec((bm, n), lambda i: (i, 0)),
        ],
        out_specs=pl.BlockSpec((bm, n), lambda i: (i, 0)),
        out_shape=jax.ShapeDtypeStruct((m, n), jnp.float32),
    )(a, x_bf16, ie, g1, g2)


def _sc_gather(tables, idx3d):
    n_rows, f = tables.shape
    nw, rpw, lw = idx3d.shape
    info = plsc.get_sparse_core_info()
    assert nw == info.num_cores * info.num_subcores
    mesh = plsc.VectorSubcoreMesh(core_axis_name="c", subcore_axis_name="s")

    @functools.partial(
        pl.kernel,
        out_type=jax.ShapeDtypeStruct((nw * rpw * lw, f), jnp.float32),
        mesh=mesh,
        scratch_types=[
            pltpu.VMEM((rpw, lw), jnp.int32),
            pltpu.VMEM((rpw * lw, f), jnp.float32),
            pltpu.SemaphoreType.DMA,
        ],
    )
    def gather_k(tab_ref, idx_ref, out_ref, idx_v, rows_v, sem):
        wid = lax.axis_index("s") * info.num_cores + lax.axis_index("c")
        pltpu.sync_copy(idx_ref.at[wid], idx_v)
        cps = [
            pltpu.async_copy(
                tab_ref.at[idx_v.at[j]],
                rows_v.at[pl.ds(j * lw, lw)],
                sem,
            )
            for j in range(rpw)
        ]
        for c in cps:
            c.wait()
        pltpu.sync_copy(rows_v, out_ref.at[pl.ds(wid * rpw * lw, rpw * lw)])

    return gather_k(tables, idx3d)


def _loss_body(u_ref, i_ref, j_ref, pi_ref, pj_ref, loss_ref, loss2_ref):
    u = u_ref[...]
    ie = i_ref[...]
    je = j_ref[...]
    pi = jnp.sum(u * ie, axis=1)
    pj = jnp.sum(u * je, axis=1)
    pi_ref[...] = pi
    pj_ref[...] = pj
    d = pi - pj
    loss2 = jnp.mean(jnp.log(1.0 + jnp.exp(-d)))
    l2 = 0.0001 * jnp.sum(u * u + ie * ie + je * je, axis=1)
    loss2_ref[...] = jnp.reshape(loss2, (1, 1))
    loss_ref[...] = jnp.reshape(loss2 + jnp.mean(l2), (1, 1))


def _loss(u, i_e, j_e):
    b, f = u.shape
    return pl.pallas_call(
        _loss_body,
        out_shape=(
            jax.ShapeDtypeStruct((b,), jnp.float32),
            jax.ShapeDtypeStruct((b,), jnp.float32),
            jax.ShapeDtypeStruct((1, 1), jnp.float32),
            jax.ShapeDtypeStruct((1, 1), jnp.float32),
        ),
    )(u, i_e, j_e)


@jax.jit
def kernel(user, item_i, item_j, user_item_3, item_user_3, user_js,
           embed_user_weight, embed_item_weight,
           user_item_matrix, item_user_matrix):
    ue = embed_user_weight
    ie = embed_item_weight
    u_n, f = ue.shape
    i_n = ie.shape[0]
    b = user.shape[0]

    ue_b = ue.astype(jnp.bfloat16)
    ie_b = ie.astype(jnp.bfloat16)

    g1i = _mm(item_user_matrix, ue_b)
    x2 = jnp.concatenate([ie, g1i], axis=1).astype(jnp.bfloat16)
    g12u = _mm(user_item_matrix, x2)
    g1u = g12u[:, :f]
    g2u = g12u[:, f:]
    gcn_users = _mm_users(user_item_3, ie_b, ue, g1u, g2u, user_js)
    g2i = _mm(item_user_matrix, g1u.astype(jnp.bfloat16))
    gcn_items = _mm_items(item_user_3, ue_b, ie, g1i, g2i)

    tables = jnp.concatenate([gcn_users, gcn_items], axis=0)
    idx = jnp.concatenate(
        [user.astype(jnp.int32),
         item_i.astype(jnp.int32) + u_n,
         item_j.astype(jnp.int32) + u_n]).reshape(32, -1, 128)
    rows = _sc_gather(tables, idx)
    u_e = rows[:b]
    i_e = rows[b:2 * b]
    j_e = rows[2 * b:]

    pi, pj, loss, loss2 = _loss(u_e, i_e, j_e)
    return pi, pj, loss[0, 0], loss2[0, 0]
